# Initial kernel scaffold; baseline (speedup 1.0000x reference)
import functools

import jax
import jax.numpy as jnp
from jax import lax
from jax.experimental import pallas as pl
from jax.experimental.pallas import tpu as pltpu
from jax.experimental.pallas import tpu_sc as plsc

_BN_EPS = 1e-5
_NC = 2
_NS = 16
_NW = _NC * _NS
_CHUNK = 128


def _make_sc_agg(n_pad, d, k_chunks):
    rpt = n_pad // _NS
    mesh = plsc.VectorSubcoreMesh(core_axis_name="c", subcore_axis_name="s")

    @functools.partial(
        pl.kernel,
        out_type=jax.ShapeDtypeStruct((_NC * n_pad, d), jnp.float32),
        mesh=mesh,
        scratch_types=[
            pltpu.VMEM((k_chunks, _CHUNK), jnp.int32),
            pltpu.VMEM((k_chunks, _CHUNK), jnp.int32),
            pltpu.VMEM((_CHUNK, d), jnp.float32),
            pltpu.VMEM_SHARED((n_pad, d), jnp.float32),
            pltpu.SemaphoreType.DMA,
        ],
    )
    def agg_kernel(h_hbm, src_hbm, dst_hbm, out_hbm, src_v, dst_v, rows_v,
                   acc_sh, sem):
        cid = lax.axis_index("c")
        sid = lax.axis_index("s")
        wid = sid * _NC + cid

        def zbody(t, carry):
            rows_v[t // (d // 16), pl.ds((t % (d // 16)) * 16, 16)] = (
                jnp.zeros((16,), jnp.float32))
            return carry
        lax.fori_loop(0, _CHUNK * (d // 16), zbody, 0)

        base = sid * rpt
        off = 0
        left = rpt
        while left > 0:
            c = min(_CHUNK, left)
            pltpu.sync_copy(rows_v.at[pl.ds(0, c)],
                            acc_sh.at[pl.ds(base + off, c)])
            off += c
            left -= c
        plsc.subcore_barrier()

        pltpu.sync_copy(src_hbm.at[wid], src_v)
        pltpu.sync_copy(dst_hbm.at[wid], dst_v)

        def body(j, carry):
            pltpu.async_copy(h_hbm.at[src_v.at[j]], rows_v, sem).wait()
            pltpu.sync_copy(rows_v, acc_sh.at[dst_v.at[j]], add=True)
            return carry
        lax.fori_loop(0, k_chunks, body, 0)

        plsc.subcore_barrier()
        pltpu.sync_copy(acc_sh.at[pl.ds(base, rpt)],
                        out_hbm.at[pl.ds(cid * n_pad + base, rpt)])

    return agg_kernel


def _enc_body(g, x_ref, bidx_ref, w_ref, b_ref, fc_ref, h_ref, z_ref, acc):
    i = pl.program_id(0)
    xb = x_ref[...]
    h_ref[...] = jnp.dot(xb, w_ref[...],
                         preferred_element_type=jnp.float32) + b_ref[...]
    oh = (bidx_ref[...] == lax.broadcasted_iota(jnp.int32, (1, g), 1)
          ).astype(jnp.float32)
    p = lax.dot_general(oh, xb, (((0,), (0,)), ((), ())),
                        preferred_element_type=jnp.float32)

    @pl.when(i == 0)
    def _():
        acc[...] = p

    @pl.when(i > 0)
    def _():
        acc[...] += p

    @pl.when(i == pl.num_programs(0) - 1)
    def _():
        z_ref[...] = jnp.dot(acc[...], fc_ref[...],
                             preferred_element_type=jnp.float32)


def _layer_body(g, h_ref, p_ref, bidx_ref, w1_ref, b1_ref, w2_ref, b2_ref,
                fc_ref, hout_ref, z_ref, acc):
    i = pl.program_id(0)
    y = h_ref[...] + p_ref[0] + p_ref[1]
    m = jnp.maximum(
        jnp.dot(y, w1_ref[...], preferred_element_type=jnp.float32)
        + b1_ref[...], 0.0)
    h2 = jnp.maximum(
        jnp.dot(m, w2_ref[...], preferred_element_type=jnp.float32)
        + b2_ref[...], 0.0)
    hout_ref[...] = h2
    oh = (bidx_ref[...] == lax.broadcasted_iota(jnp.int32, (1, g), 1)
          ).astype(jnp.float32)
    p = lax.dot_general(oh, h2, (((0,), (0,)), ((), ())),
                        preferred_element_type=jnp.float32)

    @pl.when(i == 0)
    def _():
        acc[...] = p

    @pl.when(i > 0)
    def _():
        acc[...] += p

    @pl.when(i == pl.num_programs(0) - 1)
    def _():
        z_ref[...] = jnp.dot(acc[...], fc_ref[...],
                             preferred_element_type=jnp.float32)


def kernel(x, edge_index, batch_idx, y, W_enc, b_enc, conv_W1, conv_b1,
           conv_bng, conv_bnb, conv_W2, conv_b2, bn_g, bn_b,
           fc0_W, fc0_b, fc_W, fc_b):
    n, nf = x.shape
    d = W_enc.shape[1]
    num_layers = conv_W1.shape[0]
    g = y.shape[0]
    c = fc0_W.shape[1]
    e = edge_index.shape[1]

    n_pad = ((n + 1 + _NW - 1) // _NW) * _NW
    blk = n_pad
    for cand in (4, 8, 16, 2, 32, 1):
        if n_pad % cand == 0 and (n_pad // cand) % 8 == 0:
            blk = n_pad // cand
            break
    grid_n = n_pad // blk

    k_chunks = (e + _NW * _CHUNK - 1) // (_NW * _CHUNK)
    e_pad = _NW * _CHUNK * k_chunks
    pad = jnp.full((e_pad - e,), n, jnp.int32)
    src_r = jnp.concatenate([edge_index[0], pad]).reshape(_NW, k_chunks, _CHUNK)
    dst_r = jnp.concatenate([edge_index[1], pad]).reshape(_NW, k_chunks, _CHUNK)

    x_pad = jnp.zeros((n_pad, nf), jnp.float32).at[:n].set(x)
    bidx_pad = jnp.full((n_pad, 1), g, jnp.int32).at[:n, 0].set(batch_idx)

    inv = 1.0 / jnp.sqrt(jnp.float32(1.0 + _BN_EPS))
    s1 = conv_bng * inv
    w1p = conv_W1 * s1[:, None, :]
    b1p = (conv_b1 * s1 + conv_bnb).reshape(num_layers, 1, d)
    s2 = bn_g * inv
    w2p = conv_W2 * s2[:, None, :]
    b2p = (conv_b2 * s2 + bn_b).reshape(num_layers, 1, d)
    benc = b_enc.reshape(1, d)
    fc0p = jnp.zeros((nf, 128), jnp.float32).at[:, :c].set(fc0_W)
    fcp = jnp.zeros((num_layers, d, 128), jnp.float32).at[:, :, :c].set(fc_W)

    row_spec = pl.BlockSpec((blk, d), lambda i: (i, 0))
    bidx_spec = pl.BlockSpec((blk, 1), lambda i: (i, 0))
    full = lambda shape: pl.BlockSpec(shape, lambda i: tuple(0 for _ in shape))

    enc_call = pl.pallas_call(
        functools.partial(_enc_body, g),
        grid=(grid_n,),
        in_specs=[row_spec, bidx_spec, full((nf, d)), full((1, d)),
                  full((nf, 128))],
        out_specs=[row_spec, full((g, 128))],
        out_shape=[jax.ShapeDtypeStruct((n_pad, d), jnp.float32),
                   jax.ShapeDtypeStruct((g, 128), jnp.float32)],
        scratch_shapes=[pltpu.VMEM((g, nf), jnp.float32)],
    )
    h, z = enc_call(x_pad, bidx_pad, W_enc, benc, fc0p)

    layer_call = pl.pallas_call(
        functools.partial(_layer_body, g),
        grid=(grid_n,),
        in_specs=[row_spec, pl.BlockSpec((2, blk, d), lambda i: (0, i, 0)),
                  bidx_spec, full((d, d)), full((1, d)), full((d, d)),
                  full((1, d)), full((d, 128))],
        out_specs=[row_spec, full((g, 128))],
        out_shape=[jax.ShapeDtypeStruct((n_pad, d), jnp.float32),
                   jax.ShapeDtypeStruct((g, 128), jnp.float32)],
        scratch_shapes=[pltpu.VMEM((g, d), jnp.float32)],
    )

    agg_call = _make_sc_agg(n_pad, d, k_chunks)

    for i in range(num_layers):
        parts = agg_call(h, src_r, dst_r).reshape(_NC, n_pad, d)
        h, zi = layer_call(h, parts, bidx_pad, w1p[i], b1p[i], w2p[i],
                           b2p[i], fcp[i])
        z = z + zi

    out = z[:, :c] + fc0_b + fc_b.sum(axis=0)
    return (out, y)


# speedup vs baseline: 4.4000x; 4.4000x over previous
import functools

import jax
import jax.numpy as jnp
from jax import lax
from jax.experimental import pallas as pl
from jax.experimental.pallas import tpu as pltpu
from jax.experimental.pallas import tpu_sc as plsc

_BN_EPS = 1e-5
_NC = 2
_NS = 16
_NW = _NC * _NS
_CHUNK = 128


def _make_sc_agg(n_pad, d, k_chunks):
    rpt = n_pad // _NS
    mesh = plsc.VectorSubcoreMesh(core_axis_name="c", subcore_axis_name="s")

    @functools.partial(
        pl.kernel,
        out_type=jax.ShapeDtypeStruct((_NC * n_pad, d), jnp.float32),
        mesh=mesh,
        scratch_types=[
            pltpu.VMEM((k_chunks, _CHUNK), jnp.int32),
            pltpu.VMEM((k_chunks, _CHUNK), jnp.int32),
            pltpu.VMEM((_CHUNK, d), jnp.float32),
            pltpu.VMEM_SHARED((n_pad, d), jnp.float32),
            pltpu.SemaphoreType.DMA,
        ],
    )
    def agg_kernel(h_hbm, src_hbm, dst_hbm, out_hbm, src_v, dst_v, rows_v,
                   acc_sh, sem):
        cid = lax.axis_index("c")
        sid = lax.axis_index("s")
        wid = sid * _NC + cid

        def zbody(t, carry):
            rows_v[t // (d // 16), pl.ds((t % (d // 16)) * 16, 16)] = (
                jnp.zeros((16,), jnp.float32))
            return carry
        lax.fori_loop(0, _CHUNK * (d // 16), zbody, 0)

        base = sid * rpt
        off = 0
        left = rpt
        while left > 0:
            c = min(_CHUNK, left)
            pltpu.sync_copy(rows_v.at[pl.ds(0, c)],
                            acc_sh.at[pl.ds(base + off, c)])
            off += c
            left -= c
        plsc.subcore_barrier()

        pltpu.sync_copy(src_hbm.at[wid], src_v)
        pltpu.sync_copy(dst_hbm.at[wid], dst_v)

        def body(j, carry):
            pltpu.async_copy(h_hbm.at[src_v.at[j]], rows_v, sem).wait()
            pltpu.sync_copy(rows_v, acc_sh.at[dst_v.at[j]], add=True)
            return carry
        lax.fori_loop(0, k_chunks, body, 0)

        plsc.subcore_barrier()
        pltpu.sync_copy(acc_sh.at[pl.ds(base, rpt)],
                        out_hbm.at[pl.ds(cid * n_pad + base, rpt)])

    return agg_kernel


def _enc_body(g, x_ref, bidx_ref, w_ref, b_ref, fc_ref, h_ref, z_ref, acc):
    i = pl.program_id(0)
    xb = x_ref[...]
    h_ref[...] = jnp.dot(xb, w_ref[...],
                         preferred_element_type=jnp.float32) + b_ref[...]
    oh = (bidx_ref[...] == lax.broadcasted_iota(jnp.int32, (1, g), 1)
          ).astype(jnp.float32)
    p = lax.dot_general(oh, xb, (((0,), (0,)), ((), ())),
                        preferred_element_type=jnp.float32)

    @pl.when(i == 0)
    def _():
        acc[...] = p

    @pl.when(i > 0)
    def _():
        acc[...] += p

    @pl.when(i == pl.num_programs(0) - 1)
    def _():
        z_ref[...] = jnp.dot(acc[...], fc_ref[...],
                             preferred_element_type=jnp.float32)


def _layer_body(g, h_ref, p_ref, bidx_ref, w1_ref, b1_ref, w2_ref, b2_ref,
                fc_ref, hout_ref, z_ref, acc):
    i = pl.program_id(0)
    y = h_ref[...] + p_ref[0] + p_ref[1]
    m = jnp.maximum(
        jnp.dot(y, w1_ref[...], preferred_element_type=jnp.float32)
        + b1_ref[...], 0.0)
    h2 = jnp.maximum(
        jnp.dot(m, w2_ref[...], preferred_element_type=jnp.float32)
        + b2_ref[...], 0.0)
    hout_ref[...] = h2
    oh = (bidx_ref[...] == lax.broadcasted_iota(jnp.int32, (1, g), 1)
          ).astype(jnp.float32)
    p = lax.dot_general(oh, h2, (((0,), (0,)), ((), ())),
                        preferred_element_type=jnp.float32)

    @pl.when(i == 0)
    def _():
        acc[...] = p

    @pl.when(i > 0)
    def _():
        acc[...] += p

    @pl.when(i == pl.num_programs(0) - 1)
    def _():
        z_ref[...] = jnp.dot(acc[...], fc_ref[...],
                             preferred_element_type=jnp.float32)


def kernel(x, edge_index, batch_idx, y, W_enc, b_enc, conv_W1, conv_b1,
           conv_bng, conv_bnb, conv_W2, conv_b2, bn_g, bn_b,
           fc0_W, fc0_b, fc_W, fc_b):
    n, nf = x.shape
    d = W_enc.shape[1]
    num_layers = conv_W1.shape[0]
    g = y.shape[0]
    c = fc0_W.shape[1]
    e = edge_index.shape[1]

    n_pad = ((n + 1 + 127) // 128) * 128
    blk = n_pad
    for cand in (4, 8, 16, 2, 32, 1):
        if n_pad % cand == 0 and (n_pad // cand) % 8 == 0:
            blk = n_pad // cand
            break
    grid_n = n_pad // blk

    k_chunks = (e + _NW * _CHUNK - 1) // (_NW * _CHUNK)
    e_pad = _NW * _CHUNK * k_chunks
    pad = jnp.full((e_pad - e,), n, jnp.int32)
    src_r = jnp.concatenate([edge_index[0], pad]).reshape(_NW, k_chunks, _CHUNK)
    dst_r = jnp.concatenate([edge_index[1], pad]).reshape(_NW, k_chunks, _CHUNK)

    x_pad = jnp.zeros((n_pad, nf), jnp.float32).at[:n].set(x)
    bidx_pad = jnp.full((n_pad, 1), g, jnp.int32).at[:n, 0].set(batch_idx)

    inv = 1.0 / jnp.sqrt(jnp.float32(1.0 + _BN_EPS))
    s1 = conv_bng * inv
    w1p = conv_W1 * s1[:, None, :]
    b1p = (conv_b1 * s1 + conv_bnb).reshape(num_layers, 1, d)
    s2 = bn_g * inv
    w2p = conv_W2 * s2[:, None, :]
    b2p = (conv_b2 * s2 + bn_b).reshape(num_layers, 1, d)
    benc = b_enc.reshape(1, d)
    fc0p = jnp.zeros((nf, 128), jnp.float32).at[:, :c].set(fc0_W)
    fcp = jnp.zeros((num_layers, d, 128), jnp.float32).at[:, :, :c].set(fc_W)

    row_spec = pl.BlockSpec((blk, d), lambda i: (i, 0))
    bidx_spec = pl.BlockSpec((blk, 1), lambda i: (i, 0))
    full = lambda shape: pl.BlockSpec(shape, lambda i: tuple(0 for _ in shape))

    enc_call = pl.pallas_call(
        functools.partial(_enc_body, g),
        grid=(grid_n,),
        in_specs=[row_spec, bidx_spec, full((nf, d)), full((1, d)),
                  full((nf, 128))],
        out_specs=[row_spec, full((g, 128))],
        out_shape=[jax.ShapeDtypeStruct((n_pad, d), jnp.float32),
                   jax.ShapeDtypeStruct((g, 128), jnp.float32)],
        scratch_shapes=[pltpu.VMEM((g, nf), jnp.float32)],
    )
    h, z = enc_call(x_pad, bidx_pad, W_enc, benc, fc0p)

    layer_call = pl.pallas_call(
        functools.partial(_layer_body, g),
        grid=(grid_n,),
        in_specs=[row_spec, pl.BlockSpec((2, blk, d), lambda i: (0, i, 0)),
                  bidx_spec, full((d, d)), full((1, d)), full((d, d)),
                  full((1, d)), full((d, 128))],
        out_specs=[row_spec, full((g, 128))],
        out_shape=[jax.ShapeDtypeStruct((n_pad, d), jnp.float32),
                   jax.ShapeDtypeStruct((g, 128), jnp.float32)],
        scratch_shapes=[pltpu.VMEM((g, d), jnp.float32)],
    )

    agg_call = _make_sc_agg(n_pad, d, k_chunks)

    for i in range(num_layers):
        parts = agg_call(h, src_r, dst_r).reshape(_NC, n_pad, d)
        h, zi = layer_call(h, parts, bidx_pad, w1p[i], b1p[i], w2p[i],
                           b2p[i], fcp[i])
        z = z + zi

    out = z[:, :c] + fc0_b + fc_b.sum(axis=0)
    return (out, y)
